# baseline (speedup 1.0000x reference)
import jax
import jax.numpy as jnp
from jax.experimental import pallas as pl
from jax.experimental.pallas import tpu as pltpu

_BOOK_SIZE = 1024
_LATENT = 256
_N_PTS = 576
_BLK = 512


def _transpose_kernel(in_ref, out_ref):
    out_ref[0] = jnp.transpose(in_ref[0])


def _batch_transpose(arr):
    b, m, n = arr.shape
    return pl.pallas_call(
        _transpose_kernel,
        grid=(b,),
        in_specs=[pl.BlockSpec((1, m, n), lambda i: (i, 0, 0))],
        out_specs=pl.BlockSpec((1, n, m), lambda i: (i, 0, 0)),
        out_shape=jax.ShapeDtypeStruct((b, n, m), arr.dtype),
        compiler_params=pltpu.CompilerParams(
            dimension_semantics=("arbitrary",),
        ),
    )(arr)


def _vq_block_kernel(pq_ref, x_ref, book_ref, prob_ref, logp_ref, zq_ref):
    pq = pq_ref[0, 0]
    x = x_ref[...]
    bk = book_ref[...]
    g = jax.lax.dot_general(x, bk, (((1,), (1,)), ((), ())),
                            preferred_element_type=jnp.float32)
    xx = jnp.sum(x * x, axis=1, keepdims=True)
    bb = jnp.sum(bk * bk, axis=1)[None, :]
    dist = (xx + bb) - 2.0 * g
    logits = -dist * pq
    m = jnp.max(logits, axis=1, keepdims=True)
    e = jnp.exp(logits - m)
    s = jnp.sum(e, axis=1, keepdims=True)
    prob_ref[...] = e / s
    logp_ref[...] = (logits - m) - jnp.log(s)
    iota = jax.lax.broadcasted_iota(jnp.int32, (x.shape[0], _BOOK_SIZE), 1)
    masked = jnp.where(logits == m, iota, _BOOK_SIZE)
    idx = jnp.min(masked, axis=1, keepdims=True)
    onehot = (iota == idx).astype(jnp.float32)
    zq_ref[...] = jax.lax.dot_general(onehot, bk, (((1,), (0,)), ((), ())),
                                      preferred_element_type=jnp.float32)


def kernel(ze, book, log_param_q, is_train):
    b, n_pts, latent_ndim = ze.shape
    param_q = 1.0 + jnp.exp(log_param_q)
    precision_q = 0.5 / jnp.maximum(param_q, 1e-10)
    x = _batch_transpose(ze).reshape(-1, latent_ndim)
    rows = x.shape[0]
    pq_arr = jnp.reshape(precision_q, (1, 1))
    grid = (rows // _BLK,)
    prob, logp, zq = pl.pallas_call(
        _vq_block_kernel,
        grid=grid,
        in_specs=[
            pl.BlockSpec((1, 1), lambda i: (0, 0)),
            pl.BlockSpec((_BLK, _LATENT), lambda i: (i, 0)),
            pl.BlockSpec((_BOOK_SIZE, _LATENT), lambda i: (0, 0)),
        ],
        out_specs=[
            pl.BlockSpec((_BLK, _BOOK_SIZE), lambda i: (i, 0)),
            pl.BlockSpec((_BLK, _BOOK_SIZE), lambda i: (i, 0)),
            pl.BlockSpec((_BLK, _LATENT), lambda i: (i, 0)),
        ],
        out_shape=[
            jax.ShapeDtypeStruct((rows, _BOOK_SIZE), jnp.float32),
            jax.ShapeDtypeStruct((rows, _BOOK_SIZE), jnp.float32),
            jax.ShapeDtypeStruct((rows, latent_ndim), jnp.float32),
        ],
        compiler_params=pltpu.CompilerParams(
            dimension_semantics=("parallel",),
        ),
    )(pq_arr, x, book)
    zq = _batch_transpose(zq.reshape(b, latent_ndim, n_pts))
    prob = prob.reshape(b, n_pts, _BOOK_SIZE)
    logp = logp.reshape(b, n_pts, _BOOK_SIZE)
    return (zq, precision_q, prob, logp)


# speedup vs baseline: 1.1659x; 1.1659x over previous
import jax
import jax.numpy as jnp
from jax.experimental import pallas as pl
from jax.experimental.pallas import tpu as pltpu

_BOOK_SIZE = 1024
_LATENT = 256
_N_PTS = 576
_BLK = 512


def _transpose_kernel(in_ref, out_ref):
    out_ref[...] = jnp.transpose(in_ref[...], (0, 2, 1))


_TB = 8


def _batch_transpose(arr):
    b, m, n = arr.shape
    return pl.pallas_call(
        _transpose_kernel,
        grid=(b // _TB,),
        in_specs=[pl.BlockSpec((_TB, m, n), lambda i: (i, 0, 0))],
        out_specs=pl.BlockSpec((_TB, n, m), lambda i: (i, 0, 0)),
        out_shape=jax.ShapeDtypeStruct((b, n, m), arr.dtype),
        compiler_params=pltpu.CompilerParams(
            dimension_semantics=("arbitrary",),
        ),
    )(arr)


def _vq_block_kernel(pq_ref, x_ref, book_ref, prob_ref, logp_ref, zq_ref):
    pq = pq_ref[0, 0]
    x = x_ref[...]
    bk = book_ref[...]
    g = jax.lax.dot_general(x, bk, (((1,), (1,)), ((), ())),
                            preferred_element_type=jnp.float32)
    xx = jnp.sum(x * x, axis=1, keepdims=True)
    bb = jnp.sum(bk * bk, axis=1)[None, :]
    dist = (xx + bb) - 2.0 * g
    logits = -dist * pq
    m = jnp.max(logits, axis=1, keepdims=True)
    e = jnp.exp(logits - m)
    s = jnp.sum(e, axis=1, keepdims=True)
    prob_ref[...] = e / s
    logp_ref[...] = (logits - m) - jnp.log(s)
    iota = jax.lax.broadcasted_iota(jnp.int32, (x.shape[0], _BOOK_SIZE), 1)
    masked = jnp.where(logits == m, iota, _BOOK_SIZE)
    idx = jnp.min(masked, axis=1, keepdims=True)
    onehot = (iota == idx).astype(jnp.float32)
    zq_ref[...] = jax.lax.dot_general(onehot, bk, (((1,), (0,)), ((), ())),
                                      preferred_element_type=jnp.float32)


def kernel(ze, book, log_param_q, is_train):
    b, n_pts, latent_ndim = ze.shape
    param_q = 1.0 + jnp.exp(log_param_q)
    precision_q = 0.5 / jnp.maximum(param_q, 1e-10)
    x = _batch_transpose(ze).reshape(-1, latent_ndim)
    rows = x.shape[0]
    pq_arr = jnp.reshape(precision_q, (1, 1))
    grid = (rows // _BLK,)
    prob, logp, zq = pl.pallas_call(
        _vq_block_kernel,
        grid=grid,
        in_specs=[
            pl.BlockSpec((1, 1), lambda i: (0, 0)),
            pl.BlockSpec((_BLK, _LATENT), lambda i: (i, 0)),
            pl.BlockSpec((_BOOK_SIZE, _LATENT), lambda i: (0, 0)),
        ],
        out_specs=[
            pl.BlockSpec((_BLK, _BOOK_SIZE), lambda i: (i, 0)),
            pl.BlockSpec((_BLK, _BOOK_SIZE), lambda i: (i, 0)),
            pl.BlockSpec((_BLK, _LATENT), lambda i: (i, 0)),
        ],
        out_shape=[
            jax.ShapeDtypeStruct((rows, _BOOK_SIZE), jnp.float32),
            jax.ShapeDtypeStruct((rows, _BOOK_SIZE), jnp.float32),
            jax.ShapeDtypeStruct((rows, latent_ndim), jnp.float32),
        ],
        compiler_params=pltpu.CompilerParams(
            dimension_semantics=("parallel",),
        ),
    )(pq_arr, x, book)
    zq = _batch_transpose(zq.reshape(b, latent_ndim, n_pts))
    prob = prob.reshape(b, n_pts, _BOOK_SIZE)
    logp = logp.reshape(b, n_pts, _BOOK_SIZE)
    return (zq, precision_q, prob, logp)


# speedup vs baseline: 1.1685x; 1.0023x over previous
import jax
import jax.numpy as jnp
from jax.experimental import pallas as pl
from jax.experimental.pallas import tpu as pltpu

_BOOK_SIZE = 1024
_LATENT = 256
_N_PTS = 576
_BLK = 512


def _transpose_kernel(in_ref, out_ref):
    out_ref[...] = jnp.transpose(in_ref[...], (0, 2, 1))


_TB = 8


def _batch_transpose(arr):
    b, m, n = arr.shape
    return pl.pallas_call(
        _transpose_kernel,
        grid=(b // _TB,),
        in_specs=[pl.BlockSpec((_TB, m, n), lambda i: (i, 0, 0))],
        out_specs=pl.BlockSpec((_TB, n, m), lambda i: (i, 0, 0)),
        out_shape=jax.ShapeDtypeStruct((b, n, m), arr.dtype),
        compiler_params=pltpu.CompilerParams(
            dimension_semantics=("arbitrary",),
            skip_device_barrier=True,
        ),
    )(arr)


def _vq_block_kernel(pq_ref, x_ref, book_ref, prob_ref, logp_ref, zq_ref):
    pq = pq_ref[0, 0]
    x = x_ref[...]
    bk = book_ref[...]
    g = jax.lax.dot_general(x, bk, (((1,), (1,)), ((), ())),
                            preferred_element_type=jnp.float32)
    xx = jnp.sum(x * x, axis=1, keepdims=True)
    bb = jnp.sum(bk * bk, axis=1)[None, :]
    dist = (xx + bb) - 2.0 * g
    logits = -dist * pq
    m = jnp.max(logits, axis=1, keepdims=True)
    e = jnp.exp(logits - m)
    s = jnp.sum(e, axis=1, keepdims=True)
    prob_ref[...] = e / s
    logp_ref[...] = (logits - m) - jnp.log(s)
    iota = jax.lax.broadcasted_iota(jnp.int32, (x.shape[0], _BOOK_SIZE), 1)
    masked = jnp.where(logits == m, iota, _BOOK_SIZE)
    idx = jnp.min(masked, axis=1, keepdims=True)
    onehot = (iota == idx).astype(jnp.float32)
    zq_ref[...] = jax.lax.dot_general(onehot, bk, (((1,), (0,)), ((), ())),
                                      preferred_element_type=jnp.float32)


def kernel(ze, book, log_param_q, is_train):
    b, n_pts, latent_ndim = ze.shape
    param_q = 1.0 + jnp.exp(log_param_q)
    precision_q = 0.5 / jnp.maximum(param_q, 1e-10)
    x = _batch_transpose(ze).reshape(-1, latent_ndim)
    rows = x.shape[0]
    pq_arr = jnp.reshape(precision_q, (1, 1))
    grid = (rows // _BLK,)
    prob, logp, zq = pl.pallas_call(
        _vq_block_kernel,
        grid=grid,
        in_specs=[
            pl.BlockSpec((1, 1), lambda i: (0, 0)),
            pl.BlockSpec((_BLK, _LATENT), lambda i: (i, 0)),
            pl.BlockSpec((_BOOK_SIZE, _LATENT), lambda i: (0, 0)),
        ],
        out_specs=[
            pl.BlockSpec((_BLK, _BOOK_SIZE), lambda i: (i, 0)),
            pl.BlockSpec((_BLK, _BOOK_SIZE), lambda i: (i, 0)),
            pl.BlockSpec((_BLK, _LATENT), lambda i: (i, 0)),
        ],
        out_shape=[
            jax.ShapeDtypeStruct((rows, _BOOK_SIZE), jnp.float32),
            jax.ShapeDtypeStruct((rows, _BOOK_SIZE), jnp.float32),
            jax.ShapeDtypeStruct((rows, latent_ndim), jnp.float32),
        ],
        compiler_params=pltpu.CompilerParams(
            dimension_semantics=("parallel",),
            skip_device_barrier=True,
        ),
    )(pq_arr, x, book)
    zq = _batch_transpose(zq.reshape(b, latent_ndim, n_pts))
    prob = prob.reshape(b, n_pts, _BOOK_SIZE)
    logp = logp.reshape(b, n_pts, _BOOK_SIZE)
    return (zq, precision_q, prob, logp)


# speedup vs baseline: 1.3793x; 1.1804x over previous
import jax
import jax.numpy as jnp
from jax.experimental import pallas as pl
from jax.experimental.pallas import tpu as pltpu

_BOOK_SIZE = 1024
_LATENT = 256
_N_PTS = 576
_NG = 9
_GR = 64

_X_PIECES = {
    0: [(0, 0, 256)],
    1: [(0, 256, 512)],
    2: [(0, 512, 576), (1, 0, 192)],
    3: [(1, 192, 448)],
    4: [(1, 448, 576), (2, 0, 128)],
    5: [(2, 128, 384)],
    6: [(2, 384, 576), (3, 0, 64)],
    7: [(3, 64, 320)],
    8: [(3, 320, 576)],
}
_ZQ_PIECES = {
    0: [(0, 0, 256), (1, 0, 256), (2, 0, 64)],
    1: [(2, 64, 256), (3, 0, 256), (4, 0, 128)],
    2: [(4, 128, 256), (5, 0, 256), (6, 0, 192)],
    3: [(6, 192, 256), (7, 0, 256), (8, 0, 256)],
}


def _dot(a, b_mat, dims):
    return jax.lax.dot_general(a, b_mat, (dims, ((), ())),
                               preferred_element_type=jnp.float32)


def _vq_kernel(pq_ref, ze_ref, book_ref, prob_ref, logp_ref, zq_ref):
    pq = pq_ref[0, 0]
    bk = book_ref[...]
    bb = jnp.sum(bk * bk, axis=1)[None, :]

    zeb = ze_ref[0]
    zt = jnp.transpose(zeb)

    t_iota = jax.lax.broadcasted_iota(jnp.int32, (_GR, _LATENT), 0)
    l_iota = jax.lax.broadcasted_iota(jnp.int32, (_GR, _LATENT), 1)
    sel = [(l_iota == 4 * t_iota + d).astype(jnp.float32) for d in range(4)]

    groups = []
    for i in range(_NG):
        parts = [_dot(sel[d], zt[:, q0:q1], ((1,), (0,)))
                 for (d, q0, q1) in _X_PIECES[i]]
        groups.append(parts[0] if len(parts) == 1
                      else jnp.concatenate(parts, axis=1))
    xstack = jnp.concatenate(groups, axis=0)

    r_iota = jax.lax.broadcasted_iota(jnp.int32, (_N_PTS, _N_PTS), 0)
    k_iota = jax.lax.broadcasted_iota(jnp.int32, (_N_PTS, _N_PTS), 1)
    perm = (9 * (k_iota % _GR) + k_iota // _GR == r_iota).astype(jnp.float32)
    x = _dot(perm, xstack, ((1,), (0,)))

    g = _dot(x, bk, ((1,), (1,)))
    xx = jnp.sum(x * x, axis=1, keepdims=True)
    dist = (xx + bb) - 2.0 * g
    logits = -dist * pq
    m = jnp.max(logits, axis=1, keepdims=True)
    e = jnp.exp(logits - m)
    s = jnp.sum(e, axis=1, keepdims=True)
    prob_ref[...] = e / s
    logp_ref[...] = (logits - m) - jnp.log(s)

    iota_book = jax.lax.broadcasted_iota(jnp.int32, (_N_PTS, _BOOK_SIZE), 1)
    masked = jnp.where(logits == m, iota_book, _BOOK_SIZE)
    idx = jnp.min(masked, axis=1, keepdims=True)
    onehot = (iota_book == idx).astype(jnp.float32)
    zq_flat = _dot(onehot, bk, ((1,), (0,)))

    zq_grouped = _dot(perm, zq_flat, ((0,), (0,)))
    ztq = None
    for d in range(4):
        row_d = jnp.concatenate(
            [zq_grouped[_GR * i:_GR * (i + 1), c0:c1]
             for (i, c0, c1) in _ZQ_PIECES[d]], axis=1)
        term = _dot(sel[d], row_d, ((0,), (0,)))
        ztq = term if ztq is None else ztq + term
    zq_ref[0] = jnp.transpose(ztq)


def kernel(ze, book, log_param_q, is_train):
    b, n_pts, latent_ndim = ze.shape
    param_q = 1.0 + jnp.exp(log_param_q)
    precision_q = 0.5 / jnp.maximum(param_q, 1e-10)
    pq_arr = jnp.reshape(precision_q, (1, 1))
    rows = b * n_pts
    prob, logp, zq = pl.pallas_call(
        _vq_kernel,
        grid=(b,),
        in_specs=[
            pl.BlockSpec((1, 1), lambda i: (0, 0)),
            pl.BlockSpec((1, _N_PTS, _LATENT), lambda i: (i, 0, 0)),
            pl.BlockSpec((_BOOK_SIZE, _LATENT), lambda i: (0, 0)),
        ],
        out_specs=[
            pl.BlockSpec((_N_PTS, _BOOK_SIZE), lambda i: (i, 0)),
            pl.BlockSpec((_N_PTS, _BOOK_SIZE), lambda i: (i, 0)),
            pl.BlockSpec((1, _N_PTS, _LATENT), lambda i: (i, 0, 0)),
        ],
        out_shape=[
            jax.ShapeDtypeStruct((rows, _BOOK_SIZE), jnp.float32),
            jax.ShapeDtypeStruct((rows, _BOOK_SIZE), jnp.float32),
            jax.ShapeDtypeStruct((b, n_pts, latent_ndim), jnp.float32),
        ],
        compiler_params=pltpu.CompilerParams(
            dimension_semantics=("arbitrary",),
        ),
    )(pq_arr, ze, book)
    prob = prob.reshape(b, n_pts, _BOOK_SIZE)
    logp = logp.reshape(b, n_pts, _BOOK_SIZE)
    return (zq, precision_q, prob, logp)
